# baseline (speedup 1.0000x reference)
import jax
import jax.numpy as jnp
from jax.experimental import pallas as pl
from jax.experimental.pallas import tpu as pltpu

_B = 4
_N = 20000
_NP = 20480
_K = 4096
_OUT = 512
_T = 128
_NB = _K // _T
_CAND = 768
_CTARGET = 696
_THRESH = 0.8


def _greedy_pass(width, nblocks, get_blk, x1, x2, y1, y2, area, act_ref,
                 roi_ref, rsc_ref, rlb_ref):
    eye = (jax.lax.broadcasted_iota(jnp.int32, (_T, _T), 0)
           == jax.lax.broadcasted_iota(jnp.int32, (_T, _T), 1)).astype(jnp.float32)
    low = (jax.lax.broadcasted_iota(jnp.int32, (_T, _T), 0)
           > jax.lax.broadcasted_iota(jnp.int32, (_T, _T), 1)).astype(jnp.float32)
    col_i = jax.lax.broadcasted_iota(jnp.int32, (1, width), 1)
    r_iota = jax.lax.broadcasted_iota(jnp.int32, (1, _OUT), 1).astype(jnp.float32)

    def row_to_col(r):
        return jnp.sum(eye * r, axis=1, keepdims=True)

    def block_body(carry):
        b, offs, roi_acc, sc_acc, lb_acc = carry
        off = b * _T
        blk, sblk, lblk = get_blk(off)
        xbr = blk[0:1, :]
        ybr = blk[1:2, :]
        dxb = jnp.abs(blk[3:4, :])
        dyb = jnp.abs(blk[4:5, :])
        x1c = xbr - dxb * 0.5
        x2c = xbr + dxb * 0.5
        y1c = ybr - dyb * 0.5
        y2c = ybr + dyb * 0.5
        areac = dxb * dyb
        x1r = row_to_col(x1c)
        x2r = row_to_col(x2c)
        y1r = row_to_col(y1c)
        y2r = row_to_col(y2c)
        arear = row_to_col(areac)

        ix = jnp.maximum(0.0, jnp.minimum(x2r, x2) - jnp.maximum(x1r, x1))
        iy = jnp.maximum(0.0, jnp.minimum(y2r, y2) - jnp.maximum(y1r, y1))
        inter = ix * iy
        iou = inter / jnp.maximum(arear + area - inter, 1e-6)
        s_all = (iou > _THRESH).astype(jnp.float32)

        ixb = jnp.maximum(0.0, jnp.minimum(x2r, x2c) - jnp.maximum(x1r, x1c))
        iyb = jnp.maximum(0.0, jnp.minimum(y2r, y2c) - jnp.maximum(y1r, y1c))
        interb = ixb * iyb
        ioub = interb / jnp.maximum(arear + areac - interb, 1e-6)
        m = (ioub > _THRESH).astype(jnp.float32) * low

        act_col = row_to_col(act_ref[:, pl.ds(off, _T)])

        def wcond(c):
            return c[1]

        def wbody(c):
            k, _ = c
            sup = jnp.dot(m, k, preferred_element_type=jnp.float32)
            k2 = jnp.where(sup > 0.5, 0.0, act_col)
            return (k2, jnp.any(k2 != k))

        k_col, _ = jax.lax.while_loop(wcond, wbody, (act_col, jnp.bool_(True)))

        supall = jnp.max(s_all * k_col, axis=0, keepdims=True)
        later = col_i >= off + _T
        act_ref[...] = jnp.where((supall > 0.5) & later, 0.0, act_ref[...])

        rank_col = jnp.dot(low, k_col, preferred_element_type=jnp.float32) + offs
        g = jnp.where((rank_col == r_iota) & (k_col > 0.5), 1.0, 0.0)
        roi_acc = roi_acc + jnp.dot(blk, g, preferred_element_type=jnp.float32, precision=jax.lax.Precision.HIGHEST)
        sc_acc = sc_acc + jnp.dot(sblk, g, preferred_element_type=jnp.float32, precision=jax.lax.Precision.HIGHEST)
        lb_acc = lb_acc + jnp.dot(lblk, g, preferred_element_type=jnp.float32, precision=jax.lax.Precision.HIGHEST)
        return (b + 1, offs + jnp.sum(k_col), roi_acc, sc_acc, lb_acc)

    def block_cond(carry):
        return jnp.logical_and(carry[0] < nblocks, carry[1] < float(_OUT))

    init = (jnp.int32(0),
            jnp.float32(0.0),
            jnp.zeros((7, _OUT), jnp.float32),
            jnp.zeros((1, _OUT), jnp.float32),
            jnp.zeros((1, _OUT), jnp.float32))
    _, offs, roi_acc, sc_acc, lb_acc = jax.lax.while_loop(
        block_cond, block_body, init)

    roi_ref[...] = roi_acc[None]
    rsc_ref[...] = sc_acc[None]
    rlb_ref[...] = lb_acc.astype(jnp.int32)[None] + 1
    return offs


def _nms_kernel(boxes_ref, scores_ref, labels_ref, roi_ref, rsc_ref, rlb_ref,
                active_ref):
    boxes = boxes_ref[0]
    x = boxes[0:1, :]
    y = boxes[1:2, :]
    dx = jnp.abs(boxes[3:4, :])
    dy = jnp.abs(boxes[4:5, :])
    x1 = x - dx * 0.5
    x2 = x + dx * 0.5
    y1 = y - dy * 0.5
    y2 = y + dy * 0.5
    area = dx * dy

    active_ref[...] = jnp.ones((1, _K), dtype=jnp.float32)

    def get_blk(off):
        blk = boxes_ref[0, :, pl.ds(off, _T)]
        sblk = scores_ref[0, :, pl.ds(off, _T)]
        lblk = labels_ref[0, :, pl.ds(off, _T)].astype(jnp.float32)
        return blk, sblk, lblk

    _greedy_pass(_K, _NB, get_blk, x1, x2, y1, y2, area, active_ref,
                 roi_ref, rsc_ref, rlb_ref)


def _fast_kernel(boxes_ref, scores_ref, labels_ref,
                 roi_ref, rsc_ref, rlb_ref, ok_ref, sort_ref, act_ref):
    s = scores_ref[0]
    np_iota = jax.lax.broadcasted_iota(jnp.int32, (1, _NP), 1)
    valid = np_iota < _N
    big = jnp.float32(3.0e38)

    hi0 = jnp.max(jnp.where(valid, s, -big))
    lo0 = jnp.min(jnp.where(valid, s, big)) - 1.0

    def bis(i, c):
        lo, hi = c
        mid = 0.5 * (lo + hi)
        cmid = jnp.sum(jnp.where(valid & (s > mid), 1.0, 0.0))
        p = cmid >= float(_CTARGET)
        return (jnp.where(p, mid, lo), jnp.where(p, hi, mid))

    tau, _ = jax.lax.fori_loop(0, 48, bis, (lo0, hi0))
    cnt = jnp.sum(jnp.where(valid & (s > tau), 1.0, 0.0))
    ok_a = cnt <= float(_CAND)

    eye = (jax.lax.broadcasted_iota(jnp.int32, (_T, _T), 0)
           == jax.lax.broadcasted_iota(jnp.int32, (_T, _T), 1)).astype(jnp.float32)
    low = (jax.lax.broadcasted_iota(jnp.int32, (_T, _T), 0)
           > jax.lax.broadcasted_iota(jnp.int32, (_T, _T), 1)).astype(jnp.float32)
    l_iota = jax.lax.broadcasted_iota(jnp.int32, (1, _T), 1)
    rc_iota = jax.lax.broadcasted_iota(jnp.int32, (1, _CAND), 1).astype(jnp.float32)

    def comp_body(bb, c):
        offs, acc = c
        off = bb * _T
        sb = scores_ref[0, :, pl.ds(off, _T)]
        vblk = (off + l_iota) < _N
        mrow = jnp.where(vblk & (sb > tau), 1.0, 0.0)
        mcol = jnp.sum(eye * mrow, axis=1, keepdims=True)
        rank = jnp.dot(low, mcol, preferred_element_type=jnp.float32) + offs
        g = jnp.where((rank == rc_iota) & (mcol > 0.5), 1.0, 0.0)
        bblk = boxes_ref[0, :, pl.ds(off, _T)]
        lblk = labels_ref[0, :, pl.ds(off, _T)].astype(jnp.float32)
        data = jnp.concatenate([bblk, sb, lblk], axis=0)
        acc = acc + jnp.dot(data, g, preferred_element_type=jnp.float32, precision=jax.lax.Precision.HIGHEST)
        return (offs + jnp.sum(mrow), acc)

    _, acc = jax.lax.fori_loop(0, _NP // _T, comp_body,
                               (jnp.float32(0.0), jnp.zeros((9, _CAND), jnp.float32)))

    slotf = rc_iota
    srow = jnp.where(slotf < cnt, acc[7:8, :], -big)
    eyec = (jax.lax.broadcasted_iota(jnp.int32, (_CAND, _CAND), 0)
            == jax.lax.broadcasted_iota(jnp.int32, (_CAND, _CAND), 1)).astype(jnp.float32)
    scol = jnp.sum(eyec * srow, axis=1, keepdims=True)
    lane_lt_sub = (jax.lax.broadcasted_iota(jnp.int32, (_CAND, _CAND), 1)
                   < jax.lax.broadcasted_iota(jnp.int32, (_CAND, _CAND), 0))
    beats = (srow > scol) | ((srow == scol) & lane_lt_sub)
    rankc = jnp.sum(beats.astype(jnp.float32), axis=1, keepdims=True)
    perm = jnp.where(rankc == rc_iota, 1.0, 0.0)
    accf = jnp.concatenate([acc[0:7, :], srow, acc[8:9, :]], axis=0)
    sort_ref[...] = jnp.dot(accf, perm, preferred_element_type=jnp.float32, precision=jax.lax.Precision.HIGHEST)
    act_ref[...] = jnp.where(slotf < cnt, 1.0, 0.0)

    sorted_all = sort_ref[...]
    x = sorted_all[0:1, :]
    y = sorted_all[1:2, :]
    dx = jnp.abs(sorted_all[3:4, :])
    dy = jnp.abs(sorted_all[4:5, :])
    x1 = x - dx * 0.5
    x2 = x + dx * 0.5
    y1 = y - dy * 0.5
    y2 = y + dy * 0.5
    area = dx * dy

    def get_blk(off):
        blk = sort_ref[0:7, pl.ds(off, _T)]
        sblk = sort_ref[7:8, pl.ds(off, _T)]
        lblk = sort_ref[8:9, pl.ds(off, _T)]
        return blk, sblk, lblk

    offs = _greedy_pass(_CAND, _CAND // _T, get_blk, x1, x2, y1, y2, area,
                        act_ref, roi_ref, rsc_ref, rlb_ref)
    ok_b = offs >= float(_OUT)
    ok_ref[...] = jnp.where(ok_a & ok_b, 1.0, 0.0).reshape(1, 1, 1)


def kernel(batch_box_preds, batch_cls_preds):
    scores = jnp.max(batch_cls_preds, axis=-1)
    labels = jnp.argmax(batch_cls_preds, axis=-1).astype(jnp.int32)

    pad = _NP - _N
    boxes_p = jnp.pad(jnp.transpose(batch_box_preds, (0, 2, 1)),
                      ((0, 0), (0, 0), (0, pad)))
    scores_p = jnp.pad(scores, ((0, 0), (0, pad)))[:, None, :]
    labels_p = jnp.pad(labels, ((0, 0), (0, pad)))[:, None, :]

    roi_f, rsc_f, rlb_f, ok_f = pl.pallas_call(
        _fast_kernel,
        grid=(_B,),
        in_specs=[
            pl.BlockSpec((1, 7, _NP), lambda b: (b, 0, 0)),
            pl.BlockSpec((1, 1, _NP), lambda b: (b, 0, 0)),
            pl.BlockSpec((1, 1, _NP), lambda b: (b, 0, 0)),
        ],
        out_specs=[
            pl.BlockSpec((1, 7, _OUT), lambda b: (b, 0, 0)),
            pl.BlockSpec((1, 1, _OUT), lambda b: (b, 0, 0)),
            pl.BlockSpec((1, 1, _OUT), lambda b: (b, 0, 0)),
            pl.BlockSpec((1, 1, 1), lambda b: (b, 0, 0)),
        ],
        out_shape=[
            jax.ShapeDtypeStruct((_B, 7, _OUT), jnp.float32),
            jax.ShapeDtypeStruct((_B, 1, _OUT), jnp.float32),
            jax.ShapeDtypeStruct((_B, 1, _OUT), jnp.int32),
            jax.ShapeDtypeStruct((_B, 1, 1), jnp.float32),
        ],
        scratch_shapes=[
            pltpu.VMEM((9, _CAND), jnp.float32),
            pltpu.VMEM((1, _CAND), jnp.float32),
        ],
    )(boxes_p, scores_p, labels_p)

    ok_all = jnp.all(ok_f > 0.5)

    def fast_fn(_):
        return (jnp.transpose(roi_f, (0, 2, 1)), rsc_f[:, 0, :], rlb_f[:, 0, :])

    def slow_fn(_):
        top_scores, top_idx = jax.lax.top_k(scores, _K)
        top_boxes = jnp.take_along_axis(batch_box_preds, top_idx[..., None], axis=1)
        top_labels = jnp.take_along_axis(labels, top_idx, axis=1)
        boxes_tr = jnp.transpose(top_boxes, (0, 2, 1))
        roi_tr, rsc, rlb = pl.pallas_call(
            _nms_kernel,
            grid=(_B,),
            in_specs=[
                pl.BlockSpec((1, 7, _K), lambda b: (b, 0, 0)),
                pl.BlockSpec((1, 1, _K), lambda b: (b, 0, 0)),
                pl.BlockSpec((1, 1, _K), lambda b: (b, 0, 0)),
            ],
            out_specs=[
                pl.BlockSpec((1, 7, _OUT), lambda b: (b, 0, 0)),
                pl.BlockSpec((1, 1, _OUT), lambda b: (b, 0, 0)),
                pl.BlockSpec((1, 1, _OUT), lambda b: (b, 0, 0)),
            ],
            out_shape=[
                jax.ShapeDtypeStruct((_B, 7, _OUT), jnp.float32),
                jax.ShapeDtypeStruct((_B, 1, _OUT), jnp.float32),
                jax.ShapeDtypeStruct((_B, 1, _OUT), jnp.int32),
            ],
            scratch_shapes=[
                pltpu.VMEM((1, _K), jnp.float32),
            ],
        )(boxes_tr, top_scores[:, None, :], top_labels[:, None, :])
        return (jnp.transpose(roi_tr, (0, 2, 1)), rsc[:, 0, :], rlb[:, 0, :])

    return jax.lax.cond(ok_all, fast_fn, slow_fn, None)


# speedup vs baseline: 1.9613x; 1.9613x over previous
import jax
import jax.numpy as jnp
from jax.experimental import pallas as pl
from jax.experimental.pallas import tpu as pltpu

_B = 4
_N = 20000
_NP = 20480
_K = 4096
_OUT = 512
_T = 128
_NB = _K // _T
_CAND = 768
_CTARGET = 696
_THRESH = 0.8


def _greedy_pass(width, nblocks, get_blk, x1, x2, y1, y2, area, act_ref,
                 roi_ref, rsc_ref, rlb_ref):
    eye = (jax.lax.broadcasted_iota(jnp.int32, (_T, _T), 0)
           == jax.lax.broadcasted_iota(jnp.int32, (_T, _T), 1)).astype(jnp.float32)
    low = (jax.lax.broadcasted_iota(jnp.int32, (_T, _T), 0)
           > jax.lax.broadcasted_iota(jnp.int32, (_T, _T), 1)).astype(jnp.float32)
    col_i = jax.lax.broadcasted_iota(jnp.int32, (1, width), 1)
    r_iota = jax.lax.broadcasted_iota(jnp.int32, (1, _OUT), 1).astype(jnp.float32)

    def row_to_col(r):
        return jnp.sum(eye * r, axis=1, keepdims=True)

    def block_body(carry):
        b, offs, roi_acc, sc_acc, lb_acc = carry
        off = b * _T
        blk, sblk, lblk = get_blk(off)
        xbr = blk[0:1, :]
        ybr = blk[1:2, :]
        dxb = jnp.abs(blk[3:4, :])
        dyb = jnp.abs(blk[4:5, :])
        x1c = xbr - dxb * 0.5
        x2c = xbr + dxb * 0.5
        y1c = ybr - dyb * 0.5
        y2c = ybr + dyb * 0.5
        areac = dxb * dyb
        x1r = row_to_col(x1c)
        x2r = row_to_col(x2c)
        y1r = row_to_col(y1c)
        y2r = row_to_col(y2c)
        arear = row_to_col(areac)

        ix = jnp.maximum(0.0, jnp.minimum(x2r, x2) - jnp.maximum(x1r, x1))
        iy = jnp.maximum(0.0, jnp.minimum(y2r, y2) - jnp.maximum(y1r, y1))
        inter = ix * iy
        iou = inter / jnp.maximum(arear + area - inter, 1e-6)
        s_all = (iou > _THRESH).astype(jnp.float32)

        ixb = jnp.maximum(0.0, jnp.minimum(x2r, x2c) - jnp.maximum(x1r, x1c))
        iyb = jnp.maximum(0.0, jnp.minimum(y2r, y2c) - jnp.maximum(y1r, y1c))
        interb = ixb * iyb
        ioub = interb / jnp.maximum(arear + areac - interb, 1e-6)
        m = (ioub > _THRESH).astype(jnp.float32) * low

        act_col = row_to_col(act_ref[:, pl.ds(off, _T)])

        def wcond(c):
            return c[1]

        def wbody(c):
            k, _ = c
            sup = jnp.dot(m, k, preferred_element_type=jnp.float32)
            k2 = jnp.where(sup > 0.5, 0.0, act_col)
            return (k2, jnp.any(k2 != k))

        k_col, _ = jax.lax.while_loop(wcond, wbody, (act_col, jnp.bool_(True)))

        supall = jnp.max(s_all * k_col, axis=0, keepdims=True)
        later = col_i >= off + _T
        act_ref[...] = jnp.where((supall > 0.5) & later, 0.0, act_ref[...])

        rank_col = jnp.dot(low, k_col, preferred_element_type=jnp.float32) + offs
        g = jnp.where((rank_col == r_iota) & (k_col > 0.5), 1.0, 0.0)
        roi_acc = roi_acc + jnp.dot(blk, g, preferred_element_type=jnp.float32, precision=jax.lax.Precision.HIGHEST)
        sc_acc = sc_acc + jnp.dot(sblk, g, preferred_element_type=jnp.float32, precision=jax.lax.Precision.HIGHEST)
        lb_acc = lb_acc + jnp.dot(lblk, g, preferred_element_type=jnp.float32, precision=jax.lax.Precision.HIGHEST)
        return (b + 1, offs + jnp.sum(k_col), roi_acc, sc_acc, lb_acc)

    def block_cond(carry):
        return jnp.logical_and(carry[0] < nblocks, carry[1] < float(_OUT))

    init = (jnp.int32(0),
            jnp.float32(0.0),
            jnp.zeros((7, _OUT), jnp.float32),
            jnp.zeros((1, _OUT), jnp.float32),
            jnp.zeros((1, _OUT), jnp.float32))
    _, offs, roi_acc, sc_acc, lb_acc = jax.lax.while_loop(
        block_cond, block_body, init)

    roi_ref[...] = roi_acc[None]
    rsc_ref[...] = sc_acc[None]
    rlb_ref[...] = lb_acc.astype(jnp.int32)[None] + 1
    return offs


def _nms_kernel(boxes_ref, scores_ref, labels_ref, roi_ref, rsc_ref, rlb_ref,
                active_ref):
    boxes = boxes_ref[0]
    x = boxes[0:1, :]
    y = boxes[1:2, :]
    dx = jnp.abs(boxes[3:4, :])
    dy = jnp.abs(boxes[4:5, :])
    x1 = x - dx * 0.5
    x2 = x + dx * 0.5
    y1 = y - dy * 0.5
    y2 = y + dy * 0.5
    area = dx * dy

    active_ref[...] = jnp.ones((1, _K), dtype=jnp.float32)

    def get_blk(off):
        blk = boxes_ref[0, :, pl.ds(off, _T)]
        sblk = scores_ref[0, :, pl.ds(off, _T)]
        lblk = labels_ref[0, :, pl.ds(off, _T)].astype(jnp.float32)
        return blk, sblk, lblk

    _greedy_pass(_K, _NB, get_blk, x1, x2, y1, y2, area, active_ref,
                 roi_ref, rsc_ref, rlb_ref)


def kernel(batch_box_preds, batch_cls_preds):
    scores = jnp.max(batch_cls_preds, axis=-1)
    labels = jnp.argmax(batch_cls_preds, axis=-1).astype(jnp.int32)
    top_scores, top_idx = jax.lax.top_k(scores, _K)
    top_boxes = jnp.take_along_axis(batch_box_preds, top_idx[..., None], axis=1)
    top_labels = jnp.take_along_axis(labels, top_idx, axis=1)
    boxes_tr = jnp.transpose(top_boxes, (0, 2, 1))
    roi_tr, rsc, rlb = pl.pallas_call(
        _nms_kernel,
        grid=(_B,),
        in_specs=[
            pl.BlockSpec((1, 7, _K), lambda b: (b, 0, 0)),
            pl.BlockSpec((1, 1, _K), lambda b: (b, 0, 0)),
            pl.BlockSpec((1, 1, _K), lambda b: (b, 0, 0)),
        ],
        out_specs=[
            pl.BlockSpec((1, 7, _OUT), lambda b: (b, 0, 0)),
            pl.BlockSpec((1, 1, _OUT), lambda b: (b, 0, 0)),
            pl.BlockSpec((1, 1, _OUT), lambda b: (b, 0, 0)),
        ],
        out_shape=[
            jax.ShapeDtypeStruct((_B, 7, _OUT), jnp.float32),
            jax.ShapeDtypeStruct((_B, 1, _OUT), jnp.float32),
            jax.ShapeDtypeStruct((_B, 1, _OUT), jnp.int32),
        ],
        scratch_shapes=[
            pltpu.VMEM((1, _K), jnp.float32),
        ],
    )(boxes_tr, top_scores[:, None, :], top_labels[:, None, :])
    return (jnp.transpose(roi_tr, (0, 2, 1)), rsc[:, 0, :], rlb[:, 0, :])
